# baseline (speedup 1.0000x reference)
import functools

import jax
import jax.numpy as jnp
import numpy as np
from jax import lax
from jax.experimental import pallas as pl
from jax.experimental.pallas import tpu as pltpu
from jax.experimental.pallas import tpu_sc as plsc

_NUM_FREQS = 8
_TEMPERATURE = 100.0
_GB = 16
_JOBS_PER_WORKER = 8


def _mlp_body(dr_ref, w1_ref, b1_ref, w2_ref, b2_ref, w3_ref, b3_ref,
              emb_ref, lin_ref):
    d = dr_ref[0]
    freqs = np.power(_TEMPERATURE, np.arange(_NUM_FREQS) / _NUM_FREQS)
    parts = []
    for f in freqs:
        fx = jnp.float32(f) * d
        parts.append(jnp.sin(fx))
        parts.append(jnp.cos(fx))
    e = jnp.concatenate(parts, axis=1)
    h = e
    for w_ref, b_ref in ((w1_ref, b1_ref), (w2_ref, b2_ref)):
        z = jnp.dot(h, w_ref[...], preferred_element_type=jnp.float32)
        z = z + b_ref[...]
        h = z * jax.nn.sigmoid(z)
    emb = jnp.dot(h, w3_ref[...], preferred_element_type=jnp.float32)
    emb_ref[...] = emb + b3_ref[...]

    di = d.astype(jnp.int32) // 64
    lin = di[:, 0] * 8 + di[:, 1]
    lin_ref[0, 0, :] = lin


def _sc_scatter(emb_hbm, lin_hbm, out_hbm, ebuf, lbuf, tile):
    nc = 2
    wid = lax.axis_index("s") * nc + lax.axis_index("c")

    lane = jax.lax.iota(jnp.int32, 16)
    zeros16 = jnp.zeros((16,), jnp.float32)

    def _zrow(r, _):
        row = tile.at[r]
        for k4 in range(4):
            row[pl.ds(k4 * 16, 16)] = zeros16
        return _
    lax.fori_loop(0, 512, _zrow, 0, unroll=False)

    def _job(jj, _):
        bv = wid * (_JOBS_PER_WORKER // 2) + jj // 2
        g = jj % 2
        pltpu.sync_copy(emb_hbm.at[pl.ds(bv * 40, 40), :], ebuf)
        pltpu.sync_copy(lin_hbm.at[pl.ds(bv * 40, 40)], lbuf.at[pl.ds(0, 40)])
        c0 = lbuf[pl.ds(0, 16)]
        c1 = lbuf[pl.ds(16, 16)]
        c2 = lbuf[pl.ds(32, 16)]

        def _cell_of(idx):
            z = jnp.zeros((16,), jnp.int32)
            s = jnp.where(lane == idx, c0, z)
            s = s + jnp.where(lane + 16 == idx, c1, z)
            s = s + jnp.where(lane + 32 == idx, c2, z)
            return lane * 0 + jnp.sum(s)

        for n in range(20):
            cell = _cell_of(n + 20 * g)
            for k in range(32):
                rows = lane + k * 16
                vals = ebuf[n + 20 * g, pl.ds(k * 16, 16)]
                plsc.addupdate_scatter(tile, [rows, cell], vals)
        pltpu.sync_copy(tile, out_hbm.at[pl.ds(bv * 1024 + g * 512, 512), :])
        for n in range(20):
            cell = _cell_of(n + 20 * g)
            for k in range(32):
                rows = lane + k * 16
                plsc.store_scatter(tile, [rows, cell], zeros16)
        return _
    lax.fori_loop(0, _JOBS_PER_WORKER, _job, 0, unroll=False)


@functools.partial(jax.jit, static_argnames=("interpret",))
def kernel(drags_start, drags_end, W1, b1, W2, b2, W3, b3, interpret=False):
    B, V, N, _ = drags_start.shape
    BV = B * V
    nprog = BV // _GB
    m = _GB * 40

    ds = drags_start.reshape(BV, N, 2)
    de = drags_end.reshape(BV, N, 2)
    d_rows = jnp.concatenate([ds, de], axis=1)
    dr = d_rows.reshape(nprog, m, 2)

    emb, lin = pl.pallas_call(
        _mlp_body,
        grid=(nprog,),
        in_specs=[
            pl.BlockSpec((1, m, 2), lambda i: (i, 0, 0)),
            pl.BlockSpec((32, 128), lambda i: (0, 0)),
            pl.BlockSpec((1, 128), lambda i: (0, 0)),
            pl.BlockSpec((128, 256), lambda i: (0, 0)),
            pl.BlockSpec((1, 256), lambda i: (0, 0)),
            pl.BlockSpec((256, 512), lambda i: (0, 0)),
            pl.BlockSpec((1, 512), lambda i: (0, 0)),
        ],
        out_specs=[
            pl.BlockSpec((m, 512), lambda i: (i, 0)),
            pl.BlockSpec((1, 1, m), lambda i: (i, 0, 0)),
        ],
        out_shape=[
            jax.ShapeDtypeStruct((BV * 40, 512), jnp.float32),
            jax.ShapeDtypeStruct((nprog, 1, m), jnp.int32),
        ],
        interpret=interpret,
    )(dr, W1, b1[None, :], W2, b2[None, :], W3, b3[None, :])

    lin_flat = lin.reshape(-1)

    if interpret:
        lin_jobs = lin_flat.reshape(BV * 2, 20)
        emb_jobs = emb.reshape(BV * 2, 20, 512)
        onehot = (lin_jobs[:, :, None] ==
                  jnp.arange(64, dtype=jnp.int32)[None, None, :])
        tiles = jnp.einsum("jnc,jnk->jck", emb_jobs,
                           onehot.astype(jnp.float32))
        out = tiles.reshape(BV, 2, 512, 64).reshape(BV, 1024, 64)
        return out.reshape(BV, 1024, 8, 8)

    mesh = plsc.VectorSubcoreMesh(core_axis_name="c", subcore_axis_name="s")
    sc_call = pl.kernel(
        _sc_scatter,
        mesh=mesh,
        compiler_params=pltpu.CompilerParams(needs_layout_passes=False),
        out_type=jax.ShapeDtypeStruct((BV * 1024, 64), jnp.float32),
        scratch_types=[
            pltpu.VMEM((40, 512), jnp.float32),
            pltpu.VMEM((48,), jnp.int32),
            pltpu.VMEM((512, 64), jnp.float32),
        ],
    )
    out = sc_call(emb, lin_flat)
    return out.reshape(BV, 1024, 8, 8)


# speedup vs baseline: 3.7201x; 3.7201x over previous
import functools

import jax
import jax.numpy as jnp
import numpy as np
from jax.experimental import pallas as pl

_NUM_FREQS = 8
_TEMPERATURE = 100.0
_GB = 32


def _fused_body(xt_ref, dr_ref, w1t_ref, b1_ref, w2t_ref, b2_ref, w3t_ref,
                b3_ref, out_ref):
    xt = xt_ref[0]
    m = xt.shape[1]

    freqs = np.power(_TEMPERATURE, np.arange(_NUM_FREQS) / _NUM_FREQS)
    parts = []
    for f in freqs:
        fx = jnp.float32(f) * xt
        parts.append(jnp.sin(fx))
        parts.append(jnp.cos(fx))
    et = jnp.concatenate(parts, axis=0)

    h = et
    for wt_ref, b_ref in ((w1t_ref, b1_ref), (w2t_ref, b2_ref)):
        z = jnp.dot(wt_ref[...], h, preferred_element_type=jnp.float32)
        z = z + b_ref[...]
        h = z * jax.nn.sigmoid(z)
    embt = jnp.dot(w3t_ref[...], h, preferred_element_type=jnp.float32)
    embt = embt + b3_ref[...]

    di = dr_ref[0].astype(jnp.int32) // 64
    lin = di[:, 0:1] * 8 + di[:, 1:2]
    cells = jax.lax.broadcasted_iota(jnp.int32, (m, 64), 1)
    p = (lin == cells).astype(jnp.float32)

    for j in range(_GB):
        for g in range(2):
            c0 = j * 40 + g * 20
            tile = jnp.dot(embt[:, c0:c0 + 20], p[c0:c0 + 20, :],
                           preferred_element_type=jnp.float32)
            out_ref[j, g * 512:(g + 1) * 512, :] = tile


@functools.partial(jax.jit, static_argnames=("interpret",))
def kernel(drags_start, drags_end, W1, b1, W2, b2, W3, b3, interpret=False):
    B, V, N, _ = drags_start.shape
    BV = B * V
    nprog = BV // _GB

    ds = drags_start.reshape(BV, N, 2)
    de = drags_end.reshape(BV, N, 2)
    d_rows = jnp.concatenate([ds, de], axis=1)
    xt = d_rows.transpose(0, 2, 1).reshape(nprog, _GB, 2, 40)
    xt = xt.transpose(0, 2, 1, 3).reshape(nprog, 2, _GB * 40)
    dr = d_rows.reshape(nprog, _GB * 40, 2)

    out = pl.pallas_call(
        _fused_body,
        grid=(nprog,),
        in_specs=[
            pl.BlockSpec((1, 2, _GB * 40), lambda i: (i, 0, 0)),
            pl.BlockSpec((1, _GB * 40, 2), lambda i: (i, 0, 0)),
            pl.BlockSpec((128, 32), lambda i: (0, 0)),
            pl.BlockSpec((128, 1), lambda i: (0, 0)),
            pl.BlockSpec((256, 128), lambda i: (0, 0)),
            pl.BlockSpec((256, 1), lambda i: (0, 0)),
            pl.BlockSpec((512, 256), lambda i: (0, 0)),
            pl.BlockSpec((512, 1), lambda i: (0, 0)),
        ],
        out_specs=pl.BlockSpec((_GB, 1024, 64), lambda i: (i, 0, 0)),
        out_shape=jax.ShapeDtypeStruct((BV, 1024, 64), jnp.float32),
        interpret=interpret,
    )(xt, dr, W1.T, b1[:, None], W2.T, b2[:, None], W3.T, b3[:, None])
    return out.reshape(BV, 1024, 8, 8)
